# baseline (speedup 1.0000x reference)
import functools

import jax
import jax.numpy as jnp
from jax import lax
from jax.experimental import pallas as pl
from jax.experimental.pallas import tpu as pltpu
from jax.experimental.pallas import tpu_sc as plsc

NC = 2
NS = 16
LANES = 16
IDX_CHUNK = 100
NBUF = 3


def _emb_kernel_body(S, D, BPW, seq_hbm, table_hbm, pos_hbm, out_hbm,
                     idx_all, pos_v, *bufs_and_sems):
    rows = bufs_and_sems[:NBUF]
    gsem = bufs_and_sems[NBUF:2 * NBUF]
    osem = bufs_and_sems[2 * NBUF:3 * NBUF]

    wid = lax.axis_index("s") * NC + lax.axis_index("c")
    nch = S // IDX_CHUNK
    base = wid * BPW

    pltpu.sync_copy(pos_hbm.at[pl.ds(0, S)], pos_v)
    pltpu.sync_copy(seq_hbm.at[pl.ds(base * nch, BPW * nch)], idx_all)

    def g_issue(t):
        b = t % NBUF
        return [
            pltpu.async_copy(
                table_hbm.at[idx_all.at[t * nch + c]],
                rows[b].at[pl.ds(c * IDX_CHUNK, IDX_CHUNK)],
                gsem[b],
            )
            for c in range(nch)
        ]

    OSPLIT = 104

    def o_issue_lo(t):
        b = t % NBUF
        return pltpu.async_copy(
            rows[b].at[pl.ds(0, OSPLIT)],
            out_hbm.at[pl.ds((base + t) * S, OSPLIT)],
            osem[b],
        )

    def o_issue_hi(t):
        b = t % NBUF
        return pltpu.async_copy(
            rows[b].at[pl.ds(OSPLIT, S - OSPLIT)],
            out_hbm.at[pl.ds((base + t) * S + OSPLIT, S - OSPLIT)],
            osem[b],
        )

    def add_pos_range(t, lo, hi):
        b = t % NBUF

        def s_body(k, c2):
            for u in range(2):
                s2 = 2 * k + u
                for j in range(D // LANES):
                    sl = pl.ds(j * LANES, LANES)
                    plsc.addupdate(rows[b].at[s2, sl], pos_v[s2, sl])
            return c2

        lax.fori_loop(lo // 2, hi // 2, s_body, 0)

    g = {t: g_issue(t) for t in range(NBUF)}
    o = {}
    for t in range(BPW):
        for cp in g[t]:
            cp.wait()
        add_pos_range(t, 0, OSPLIT)
        o_lo = o_issue_lo(t)
        add_pos_range(t, OSPLIT, S)
        o[t] = (o_lo, o_issue_hi(t))
        if t >= 1 and t + 2 < BPW:
            for cp in o[t - 1]:
                cp.wait()
            g[t + 2] = g_issue(t + 2)
    for t in (BPW - 3, BPW - 2, BPW - 1):
        for cp in o[t]:
            cp.wait()


def kernel(sequence, token_weight, position_weight):
    B, S = sequence.shape
    V, D = token_weight.shape
    NW = NC * NS
    BPW = B // NW
    nch = S // IDX_CHUNK

    seq = sequence.astype(jnp.int32).reshape(B * nch, IDX_CHUNK)

    mesh = plsc.VectorSubcoreMesh(core_axis_name="c", subcore_axis_name="s")
    body = functools.partial(_emb_kernel_body, S, D, BPW)
    out = pl.kernel(
        body,
        out_type=jax.ShapeDtypeStruct((B * S, D), jnp.float32),
        mesh=mesh,
        scratch_types=(
            [
                pltpu.VMEM((BPW * nch, IDX_CHUNK), jnp.int32),
                pltpu.VMEM((S, D), jnp.float32),
            ]
            + [pltpu.VMEM((S, D), jnp.float32)] * NBUF
            + [pltpu.SemaphoreType.DMA] * (2 * NBUF)
        ),
    )(seq, token_weight, position_weight)
    return out.reshape(B, S, D)


# speedup vs baseline: 1.0218x; 1.0218x over previous
import functools

import jax
import jax.numpy as jnp
from jax import lax
from jax.experimental import pallas as pl
from jax.experimental.pallas import tpu as pltpu
from jax.experimental.pallas import tpu_sc as plsc

NC = 2
NS = 16
LANES = 16
IDX_CHUNK = 100
NBUF = 3
NSLOT = 2
SROUTE = 72


def _emb_kernel_body(S, D, BPW, seq_hbm, table_hbm, pos_hbm, out_hbm,
                     idx_all, pos_v, shared, ssem, hsem, *bufs_and_sems):
    rows = bufs_and_sems[:NBUF]
    gsem = bufs_and_sems[NBUF:2 * NBUF]
    osem = bufs_and_sems[2 * NBUF:3 * NBUF]

    sid = lax.axis_index("s")
    wid = sid * NC + lax.axis_index("c")
    nch = S // IDX_CHUNK
    base = wid * BPW

    pltpu.sync_copy(pos_hbm.at[pl.ds(0, S)], pos_v)
    pltpu.sync_copy(seq_hbm.at[pl.ds(base * nch, BPW * nch)], idx_all)

    def g_issue(t):
        b = t % NBUF
        return [
            pltpu.async_copy(
                table_hbm.at[idx_all.at[t * nch + c]],
                rows[b].at[pl.ds(c * IDX_CHUNK, IDX_CHUNK)],
                gsem[b],
            )
            for c in range(nch)
        ]

    def o_issue(t):
        b = t % NBUF
        return pltpu.async_copy(
            rows[b].at[pl.ds(SROUTE, S - SROUTE)],
            out_hbm.at[pl.ds((base + t) * S + SROUTE, S - SROUTE)],
            osem[b],
        )

    def slot(t):
        return shared.at[sid, t % NSLOT]

    def hop1_issue(t):
        return pltpu.async_copy(
            rows[t % NBUF].at[pl.ds(0, SROUTE)], slot(t), ssem
        )

    def hop2_issue(t):
        return pltpu.async_copy(
            slot(t), out_hbm.at[pl.ds((base + t) * S, SROUTE)], hsem
        )

    def add_pos(t):
        b = t % NBUF

        def s_body(k, c2):
            for u in range(2):
                s2 = 2 * k + u
                for j in range(D // LANES):
                    sl = pl.ds(j * LANES, LANES)
                    plsc.addupdate(rows[b].at[s2, sl], pos_v[s2, sl])
            return c2

        lax.fori_loop(0, S // 2, s_body, 0)

    g = {t: g_issue(t) for t in range(NBUF)}
    o = {}
    h1 = {}
    h2 = {}
    h2_waited = set()
    for t in range(BPW):
        for cp in g[t]:
            cp.wait()
        add_pos(t)
        prev = t - NSLOT
        if prev >= 0 and prev in h2:
            h2[prev].wait()
            h2_waited.add(prev)
        h1[t] = hop1_issue(t)
        o[t] = o_issue(t)
        if t >= 1:
            h1[t - 1].wait()
            o[t - 1].wait()
            h2[t - 1] = hop2_issue(t - 1)
            if t + 2 < BPW:
                g[t + 2] = g_issue(t + 2)
    t = BPW - 1
    h1[t].wait()
    o[t].wait()
    h2[t] = hop2_issue(t)
    for t, cp in h2.items():
        if t not in h2_waited:
            cp.wait()


def kernel(sequence, token_weight, position_weight):
    B, S = sequence.shape
    V, D = token_weight.shape
    NW = NC * NS
    BPW = B // NW
    nch = S // IDX_CHUNK

    seq = sequence.astype(jnp.int32).reshape(B * nch, IDX_CHUNK)

    mesh = plsc.VectorSubcoreMesh(core_axis_name="c", subcore_axis_name="s")
    body = functools.partial(_emb_kernel_body, S, D, BPW)
    out = pl.kernel(
        body,
        out_type=jax.ShapeDtypeStruct((B * S, D), jnp.float32),
        mesh=mesh,
        scratch_types=(
            [
                pltpu.VMEM((BPW * nch, IDX_CHUNK), jnp.int32),
                pltpu.VMEM((S, D), jnp.float32),
                pltpu.VMEM_SHARED((NS, NSLOT, SROUTE, D), jnp.float32),
                pltpu.SemaphoreType.DMA,
                pltpu.SemaphoreType.DMA,
            ]
            + [pltpu.VMEM((S, D), jnp.float32)] * NBUF
            + [pltpu.SemaphoreType.DMA] * (2 * NBUF)
        ),
    )(seq, token_weight, position_weight)
    return out.reshape(B, S, D)
